# baseline (speedup 1.0000x reference)
import functools

import jax
import jax.numpy as jnp
from jax import lax
from jax.experimental import pallas as pl
from jax.experimental.pallas import tpu as pltpu
from jax.experimental.pallas import tpu_sc as plsc

_CHUNK = 64


@functools.partial(jax.jit, static_argnames=())
def _gather_rows(idx, table):
    B = idx.shape[0]
    V, D = table.shape

    info = plsc.get_sparse_core_info()
    num_workers = info.num_cores * info.num_subcores
    b_per_w = B // num_workers
    n_chunks = b_per_w // _CHUNK

    mesh = plsc.VectorSubcoreMesh(core_axis_name="c", subcore_axis_name="s")

    @functools.partial(
        pl.kernel,
        mesh=mesh,
        out_type=jax.ShapeDtypeStruct((B, D), jnp.float32),
        scratch_types=[
            pltpu.VMEM((b_per_w,), jnp.int32),
            pltpu.VMEM((2, _CHUNK, D), jnp.float32),
            pltpu.SemaphoreType.DMA,
        ],
    )
    def k(idx_hbm, table_hbm, out_hbm, idx_v, rows_v, gsem):
        wid = lax.axis_index("s") * info.num_cores + lax.axis_index("c")
        base = wid * b_per_w
        pltpu.sync_copy(idx_hbm.at[pl.ds(base, b_per_w)], idx_v)

        gathers = [None, None]
        gathers[0] = pltpu.async_copy(
            table_hbm.at[idx_v.at[pl.ds(0, _CHUNK)]], rows_v.at[0], gsem
        )
        for g in range(n_chunks):
            buf = g % 2
            gathers[buf].wait()
            if g + 1 < n_chunks:
                gathers[1 - buf] = pltpu.async_copy(
                    table_hbm.at[idx_v.at[pl.ds((g + 1) * _CHUNK, _CHUNK)]],
                    rows_v.at[1 - buf],
                    gsem,
                )
            pltpu.sync_copy(
                rows_v.at[buf], out_hbm.at[pl.ds(base + g * _CHUNK, _CHUNK)]
            )

    return k(idx, table)


def kernel(class_number, multi_vector_embedding):
    V, P, C = multi_vector_embedding.shape
    table = multi_vector_embedding.reshape(V, P * C)
    out = _gather_rows(class_number.astype(jnp.int32), table)
    return out.reshape(class_number.shape[0], P, C)


# speedup vs baseline: 15.9403x; 15.9403x over previous
import functools

import jax
import jax.numpy as jnp
from jax import lax
from jax.experimental import pallas as pl
from jax.experimental.pallas import tpu as pltpu
from jax.experimental.pallas import tpu_sc as plsc

_CHUNK = 64


@jax.jit
def _gather_rows(idx, table3):
    C, V, D = table3.shape
    B = idx.shape[0]

    info = plsc.get_sparse_core_info()
    num_workers = info.num_cores * info.num_subcores
    b_per_w = B // num_workers
    n_chunks = b_per_w // _CHUNK

    mesh = plsc.VectorSubcoreMesh(core_axis_name="c", subcore_axis_name="s")

    @functools.partial(
        pl.kernel,
        mesh=mesh,
        out_type=jax.ShapeDtypeStruct((C, B, D), jnp.float32),
        scratch_types=[
            pltpu.VMEM((b_per_w,), jnp.int32),
            pltpu.VMEM((2, _CHUNK, D), jnp.float32),
            pltpu.SemaphoreType.DMA,
        ],
    )
    def k(idx_hbm, table_hbm, out_hbm, idx_v, rows_v, gsem):
        wid = lax.axis_index("s") * info.num_cores + lax.axis_index("c")
        base = wid * b_per_w
        pltpu.sync_copy(idx_hbm.at[pl.ds(base, b_per_w)], idx_v)

        for p in range(C):
            plane = table_hbm.at[p]
            gathers = [None, None]
            gathers[0] = pltpu.async_copy(
                plane.at[idx_v.at[pl.ds(0, _CHUNK)]], rows_v.at[0], gsem
            )
            for g in range(n_chunks):
                buf = g % 2
                gathers[buf].wait()
                if g + 1 < n_chunks:
                    gathers[1 - buf] = pltpu.async_copy(
                        plane.at[idx_v.at[pl.ds((g + 1) * _CHUNK, _CHUNK)]],
                        rows_v.at[1 - buf],
                        gsem,
                    )
                pltpu.sync_copy(
                    rows_v.at[buf],
                    out_hbm.at[p].at[pl.ds(base + g * _CHUNK, _CHUNK)],
                )

    return k(idx, table3)


def kernel(class_number, multi_vector_embedding):
    table3 = jnp.transpose(multi_vector_embedding, (2, 0, 1))
    out3 = _gather_rows(class_number.astype(jnp.int32), table3)
    return jnp.transpose(out3, (1, 2, 0))


# speedup vs baseline: 19.0296x; 1.1938x over previous
import functools

import jax
import jax.numpy as jnp
from jax import lax
from jax.experimental import pallas as pl
from jax.experimental.pallas import tpu as pltpu
from jax.experimental.pallas import tpu_sc as plsc

_CHUNK = 128
_NBUF = 3


@jax.jit
def _gather_rows(idx, table3):
    C, V, D = table3.shape
    B = idx.shape[0]

    info = plsc.get_sparse_core_info()
    num_workers = info.num_cores * info.num_subcores
    b_per_w = B // num_workers
    n_chunks = b_per_w // _CHUNK

    mesh = plsc.VectorSubcoreMesh(core_axis_name="c", subcore_axis_name="s")

    @functools.partial(
        pl.kernel,
        mesh=mesh,
        out_type=jax.ShapeDtypeStruct((C, B, D), jnp.float32),
        scratch_types=[
            pltpu.VMEM((b_per_w,), jnp.int32),
            pltpu.VMEM((_NBUF, _CHUNK, D), jnp.float32),
            pltpu.SemaphoreType.DMA,
        ],
    )
    def k(idx_hbm, table_hbm, out_hbm, idx_v, rows_v, gsem):
        wid = lax.axis_index("s") * info.num_cores + lax.axis_index("c")
        base = wid * b_per_w
        pltpu.sync_copy(idx_hbm.at[pl.ds(base, b_per_w)], idx_v)

        tasks = [(p, g) for p in range(C) for g in range(n_chunks)]
        T = len(tasks)

        def gather(t):
            p, g = tasks[t]
            return pltpu.async_copy(
                table_hbm.at[p].at[idx_v.at[pl.ds(g * _CHUNK, _CHUNK)]],
                rows_v.at[t % _NBUF],
                gsem,
            )

        gathers = {t: gather(t) for t in range(_NBUF - 1)}
        for t in range(T):
            gathers[t].wait()
            if t + _NBUF - 1 < T:
                gathers[t + _NBUF - 1] = gather(t + _NBUF - 1)
            p, g = tasks[t]
            pltpu.sync_copy(
                rows_v.at[t % _NBUF],
                out_hbm.at[p].at[pl.ds(base + g * _CHUNK, _CHUNK)],
            )

    return k(idx, table3)


def kernel(class_number, multi_vector_embedding):
    table3 = jnp.transpose(multi_vector_embedding, (2, 0, 1))
    out3 = _gather_rows(class_number.astype(jnp.int32), table3)
    return jnp.transpose(out3, (1, 2, 0))


# speedup vs baseline: 19.0706x; 1.0022x over previous
import functools

import jax
import jax.numpy as jnp
from jax import lax
from jax.experimental import pallas as pl
from jax.experimental.pallas import tpu as pltpu
from jax.experimental.pallas import tpu_sc as plsc

_CHUNK = 128
_NBUF = 3


@jax.jit
def _gather_rows(idx, table3):
    C, V, D = table3.shape
    B = idx.shape[0]

    info = plsc.get_sparse_core_info()
    num_workers = info.num_cores * info.num_subcores
    b_per_w = B // num_workers
    n_chunks = b_per_w // _CHUNK

    mesh = plsc.VectorSubcoreMesh(core_axis_name="c", subcore_axis_name="s")

    @functools.partial(
        pl.kernel,
        mesh=mesh,
        out_type=jax.ShapeDtypeStruct((C, B, D), jnp.float32),
        scratch_types=[
            pltpu.VMEM((b_per_w,), jnp.int32),
            pltpu.VMEM((_NBUF, _CHUNK, D), jnp.float32),
            pltpu.SemaphoreType.DMA,
            pltpu.SemaphoreType.DMA,
        ],
    )
    def k(idx_hbm, table_hbm, out_hbm, idx_v, rows_v, gsem, ssem):
        wid = lax.axis_index("s") * info.num_cores + lax.axis_index("c")
        base = wid * b_per_w
        pltpu.sync_copy(idx_hbm.at[pl.ds(base, b_per_w)], idx_v)

        tasks = [(p, g) for p in range(C) for g in range(n_chunks)]
        T = len(tasks)

        def gather(t):
            p, g = tasks[t]
            return pltpu.async_copy(
                table_hbm.at[p].at[idx_v.at[pl.ds(g * _CHUNK, _CHUNK)]],
                rows_v.at[t % _NBUF],
                gsem,
            )

        def store(t):
            p, g = tasks[t]
            return pltpu.async_copy(
                rows_v.at[t % _NBUF],
                out_hbm.at[p].at[pl.ds(base + g * _CHUNK, _CHUNK)],
                ssem,
            )

        gathers = {t: gather(t) for t in range(_NBUF - 1)}
        stores = {}
        for t in range(T):
            gathers[t].wait()
            stores[t] = store(t)
            if t + _NBUF - 1 < T:
                if t - 1 >= 0:
                    stores[t - 1].wait()
                gathers[t + _NBUF - 1] = gather(t + _NBUF - 1)
        for t in range(max(0, T - _NBUF), T):
            stores[t].wait()

    return k(idx, table3)


def kernel(class_number, multi_vector_embedding):
    table3 = jnp.transpose(multi_vector_embedding, (2, 0, 1))
    out3 = _gather_rows(class_number.astype(jnp.int32), table3)
    return jnp.transpose(out3, (1, 2, 0))
